# Initial kernel scaffold; baseline (speedup 1.0000x reference)
import jax
import jax.numpy as jnp
from jax.experimental import pallas as pl


def kernel(user_table, item_table, edge_weight, edge_index, users, items):
    raise NotImplementedError("write your pallas kernel here")


# speedup vs baseline: 6.5171x; 6.5171x over previous
import functools

import jax
import jax.numpy as jnp
from jax import lax
from jax.experimental import pallas as pl
from jax.experimental.pallas import tpu as pltpu
from jax.experimental.pallas import tpu_sc as plsc

N_USERS = 50000
N_ITEMS = 50000
N_NODES = N_USERS + N_ITEMS
N_EDGES = 1600000
DIM = 32
N_LAYERS = 3
BATCH = 4096

_INFO = plsc.get_sparse_core_info()
NC = _INFO.num_cores
NS = _INFO.num_subcores
LANES = _INFO.num_lanes

HALF = N_NODES // NC
HALFP = 50176
N_PADROWS = HALFP - HALF
STRIPE = HALFP // NS
NODESP = NC * HALFP

EK = 4
EROW = 128
CHUNK = EK * EROW
ROWS_PER_TILE = 800
N_CHUNKS = ROWS_PER_TILE // EK
E_PAD = 16 * ROWS_PER_TILE * EROW

BT = BATCH // (NC * NS)

_mesh = plsc.VectorSubcoreMesh(core_axis_name="c", subcore_axis_name="s")
_cparams = pltpu.CompilerParams(needs_layout_passes=False, use_tc_tiling_on_sc=False)


def _iota16():
    return lax.iota(jnp.int32, LANES)


@functools.partial(
    pl.kernel,
    out_type=jax.ShapeDtypeStruct((NODESP, DIM), jnp.float32),
    mesh=_mesh,
    compiler_params=_cparams,
    scratch_types=[
        pltpu.VMEM((EK, EROW), jnp.int32),
        pltpu.VMEM((CHUNK,), jnp.int32),
        pltpu.VMEM((CHUNK,), jnp.int32),
        pltpu.VMEM((CHUNK,), jnp.float32),
        pltpu.VMEM((CHUNK, DIM), jnp.float32),
        pltpu.VMEM_SHARED((HALFP, DIM), jnp.float32),
        pltpu.SemaphoreType.DMA,
        pltpu.SemaphoreType.DMA,
        pltpu.SemaphoreType.DMA,
    ],
)
def _layer(table, srcm, dstf, wf, zeros, out,
           src_v, dst_v, dstl_v, w_v, rows_v, acc, sem_i, sem_g, sem_s):
    c = lax.axis_index("c")
    s = lax.axis_index("s")
    iota = _iota16()

    pltpu.sync_copy(zeros, acc.at[pl.ds(s * STRIPE, STRIPE)])
    plsc.subcore_barrier()

    lo = c * HALF

    def chunk_body(ch, carry):
        base = s * ROWS_PER_TILE + ch * EK
        ebase = base * EROW
        cp_s = pltpu.async_copy(srcm.at[pl.ds(base, EK)], src_v, sem_i)
        cp_d = pltpu.async_copy(dstf.at[pl.ds(ebase, CHUNK)], dst_v, sem_i)
        cp_w = pltpu.async_copy(wf.at[pl.ds(ebase, CHUNK)], w_v, sem_i)
        cp_s.wait()
        cp_d.wait()
        cp_w.wait()

        gathers = []
        for k in range(EK):
            gathers.append(pltpu.async_copy(
                table.at[src_v.at[k]],
                rows_v.at[pl.ds(k * EROW, EROW)], sem_g))
        for g in gathers:
            g.wait()

        def group_body(g, carry2):
            o0 = g * LANES
            dv = dst_v[pl.ds(o0, LANES)]
            dl = dv - lo
            ok = (dl >= 0) & (dl < HALF)
            dsel = jnp.where(ok, dl, HALF + ((o0 + iota) & 127))
            dstl_v[pl.ds(o0, LANES)] = dsel
            return carry2

        lax.fori_loop(0, CHUNK // LANES, group_body, 0)

        def mul_body(j, carry2):
            w16 = plsc.load_gather(w_v, [jnp.full((LANES,), j, jnp.int32)])
            r0 = rows_v[j, pl.ds(0, LANES)]
            r1 = rows_v[j, pl.ds(LANES, LANES)]
            rows_v[j, pl.ds(0, LANES)] = r0 * w16
            rows_v[j, pl.ds(LANES, LANES)] = r1 * w16
            return carry2

        lax.fori_loop(0, CHUNK, mul_body, 0)

        scats = []
        for k in range(EK):
            scats.append(pltpu.async_copy(
                rows_v.at[pl.ds(k * EROW, EROW)],
                acc.at[dstl_v.at[pl.ds(k * EROW, EROW)]], sem_s, add=True))
        for g in scats:
            g.wait()
        return carry

    lax.fori_loop(0, N_CHUNKS, chunk_body, 0)
    plsc.subcore_barrier()

    pltpu.sync_copy(acc.at[pl.ds(s * STRIPE, STRIPE)],
                    out.at[pl.ds(c * HALFP + s * STRIPE, STRIPE)])


@functools.partial(
    pl.kernel,
    out_type=jax.ShapeDtypeStruct((BATCH,), jnp.float32),
    mesh=_mesh,
    compiler_params=_cparams,
    scratch_types=[
        pltpu.VMEM((BT,), jnp.int32),
        pltpu.VMEM((BT,), jnp.int32),
        pltpu.VMEM((4 * BT, DIM), jnp.float32),
        pltpu.VMEM((4 * BT, DIM), jnp.float32),
        pltpu.VMEM((BT * DIM,), jnp.float32),
        pltpu.VMEM((BT,), jnp.float32),
        pltpu.SemaphoreType.DMA,
        pltpu.SemaphoreType.DMA,
    ],
)
def _final(e0, e1, e2, e3, users, items, out,
           u_v, i_v, ur_v, ir_v, p_v, o_v, sem_i, sem_g):
    c = lax.axis_index("c")
    s = lax.axis_index("s")
    wid = s * NC + c
    base = wid * BT
    iota = _iota16()

    cp_u = pltpu.async_copy(users.at[pl.ds(base, BT)], u_v, sem_i)
    cp_i = pltpu.async_copy(items.at[pl.ds(base, BT)], i_v, sem_i)
    cp_u.wait()
    cp_i.wait()

    for g in range(BT // LANES):
        i_v[pl.ds(g * LANES, LANES)] = i_v[pl.ds(g * LANES, LANES)] + HALFP

    gathers = []
    for t, tab in enumerate((e0, e1, e2, e3)):
        gathers.append(pltpu.async_copy(
            tab.at[u_v], ur_v.at[pl.ds(t * BT, BT)], sem_g))
        gathers.append(pltpu.async_copy(
            tab.at[i_v], ir_v.at[pl.ds(t * BT, BT)], sem_g))
    for g in gathers:
        g.wait()

    def sum_body(j, carry):
        uacc0 = jnp.zeros((LANES,), jnp.float32)
        uacc1 = jnp.zeros((LANES,), jnp.float32)
        iacc0 = jnp.zeros((LANES,), jnp.float32)
        iacc1 = jnp.zeros((LANES,), jnp.float32)
        for t in range(4):
            uacc0 = uacc0 + ur_v[t * BT + j, pl.ds(0, LANES)]
            uacc1 = uacc1 + ur_v[t * BT + j, pl.ds(LANES, LANES)]
            iacc0 = iacc0 + ir_v[t * BT + j, pl.ds(0, LANES)]
            iacc1 = iacc1 + ir_v[t * BT + j, pl.ds(LANES, LANES)]
        p_v[pl.ds(j * DIM, LANES)] = uacc0 * iacc0
        p_v[pl.ds(j * DIM + LANES, LANES)] = uacc1 * iacc1
        return carry

    lax.fori_loop(0, BT, sum_body, 0)

    for g in range(BT // LANES):
        acc = jnp.zeros((LANES,), jnp.float32)
        rowr = (g * LANES + iota) * DIM
        for d in range(DIM):
            acc = acc + plsc.load_gather(p_v, [rowr + d])
        o_v[pl.ds(g * LANES, LANES)] = acc * (1.0 / 16.0)

    pltpu.sync_copy(o_v, out.at[pl.ds(base, BT)])


def kernel(user_table, item_table, edge_weight, edge_index, users, items):
    halfpad = jnp.zeros((N_PADROWS, DIM), jnp.float32)
    table0 = jnp.concatenate([user_table, halfpad, item_table, halfpad], axis=0)

    src = edge_index[0].astype(jnp.int32)
    dst = edge_index[1].astype(jnp.int32)
    w = edge_weight.astype(jnp.float32)

    src = src + jnp.where(src >= HALF, N_PADROWS, 0).astype(jnp.int32)

    n_pad = E_PAD - N_EDGES
    pad_src = lax.iota(jnp.int32, n_pad) % N_NODES
    srcm = jnp.concatenate([src, pad_src]).reshape(-1, EROW)
    dstf = jnp.concatenate([dst, jnp.full((n_pad,), N_NODES, jnp.int32)])
    wf = jnp.concatenate([w, jnp.zeros((n_pad,), jnp.float32)])

    zeros = jnp.zeros((STRIPE, DIM), jnp.float32)

    e0 = table0
    e1 = _layer(e0, srcm, dstf, wf, zeros)
    e2 = _layer(e1, srcm, dstf, wf, zeros)
    e3 = _layer(e2, srcm, dstf, wf, zeros)

    return _final(e0, e1, e2, e3, users.astype(jnp.int32),
                  items.astype(jnp.int32))
